# baseline (speedup 1.0000x reference)
import jax
import jax.numpy as jnp
from jax import lax
from jax.experimental import pallas as pl
from jax.experimental.pallas import tpu as pltpu
from jax.experimental.pallas import tpu_sc as plsc

B = 16384
DIM = 64
PDIM = 128
NUM_CORES = 2
NUM_SUBCORES = 16
NW = NUM_CORES * NUM_SUBCORES
BPW = B // NW
CHUNK = 128
NCHUNK = BPW // CHUNK
L = 16
FGROUPS = DIM // L
BGROUPS = CHUNK // L


def _gather_body(rt_ref, ln_ref, tp_ref, w0_ref, w1_ref, w2_ref, out_ref,
                 i0a, i1a, i2a, i0b, i1b, i2b,
                 r0a, r1a, r2a, r0b, r1b, r2b,
                 outT, sem_a, sem_b):
    wid = lax.axis_index("s") * NUM_CORES + lax.axis_index("c")
    base = wid * BPW
    irefs = (rt_ref, ln_ref, tp_ref)
    tabs = (w0_ref, w1_ref, w2_ref)
    idxs = ((i0a, i1a, i2a), (i0b, i1b, i2b))
    rows = ((r0a, r1a, r2a), (r0b, r1b, r2b))
    sems = (sem_a, sem_b)

    iota = lax.iota(jnp.int32, L)
    perms = [lax.rem(iota + j, jnp.full((L,), L, jnp.int32)) for j in range(L)]

    def fire(k):
        s = k % 2
        b = base + k * CHUNK
        for c in range(3):
            pltpu.sync_copy(irefs[c].at[pl.ds(b, CHUNK)], idxs[s][c])
        return [pltpu.async_copy(tabs[c].at[idxs[s][c]], rows[s][c], sems[s])
                for c in range(3)]

    def transpose_block(src, fbase, i):
        bg = lax.div(i, FGROUPS)
        fg = lax.rem(i, FGROUPS)
        b0 = bg * L
        ridx = b0 + iota
        cidx0 = fg * L
        for j in range(L):
            cidx = cidx0 + perms[j]
            v = plsc.load_gather(src, [ridx, cidx])
            plsc.store_scatter(outT, [fbase + cidx, ridx], v)

    def process(k, descs):
        s = k % 2
        for d in descs:
            d.wait()
        for c in range(3):
            lax.fori_loop(
                0, BGROUPS * FGROUPS,
                lambda i, carry, c=c: (transpose_block(rows[s][c], DIM * c, i),
                                       carry)[1],
                0)
        b = base + k * CHUNK
        pltpu.sync_copy(outT, out_ref.at[:, pl.ds(b, CHUNK)])

    descs = fire(0)
    for k in range(NCHUNK):
        nxt = fire(k + 1) if k + 1 < NCHUNK else None
        process(k, descs)
        descs = nxt


@jax.jit
def _lookup_concat(road_type, lane, time_period, W_road, W_lane, W_time):
    pad = [(0, 0), (0, PDIM - DIM)]
    w0 = jnp.pad(W_road, pad)
    w1 = jnp.pad(W_lane, pad)
    w2 = jnp.pad(W_time, pad)

    mesh = plsc.VectorSubcoreMesh(core_axis_name="c", subcore_axis_name="s")
    out_t = pl.kernel(
        _gather_body,
        out_type=jax.ShapeDtypeStruct((3 * DIM, B), jnp.float32),
        mesh=mesh,
        compiler_params=pltpu.CompilerParams(needs_layout_passes=False),
        scratch_types=[
            pltpu.VMEM((CHUNK,), jnp.int32),
            pltpu.VMEM((CHUNK,), jnp.int32),
            pltpu.VMEM((CHUNK,), jnp.int32),
            pltpu.VMEM((CHUNK,), jnp.int32),
            pltpu.VMEM((CHUNK,), jnp.int32),
            pltpu.VMEM((CHUNK,), jnp.int32),
            pltpu.VMEM((CHUNK, PDIM), jnp.float32),
            pltpu.VMEM((CHUNK, PDIM), jnp.float32),
            pltpu.VMEM((CHUNK, PDIM), jnp.float32),
            pltpu.VMEM((CHUNK, PDIM), jnp.float32),
            pltpu.VMEM((CHUNK, PDIM), jnp.float32),
            pltpu.VMEM((CHUNK, PDIM), jnp.float32),
            pltpu.VMEM((3 * DIM, CHUNK), jnp.float32),
            pltpu.SemaphoreType.DMA,
            pltpu.SemaphoreType.DMA,
        ],
    )(road_type, lane, time_period, w0, w1, w2)
    return out_t.T


def kernel(road_type, lane, time_period, W_road, W_lane, W_time):
    return _lookup_concat(
        road_type.astype(jnp.int32),
        lane.astype(jnp.int32),
        time_period.astype(jnp.int32),
        W_road, W_lane, W_time,
    )


# speedup vs baseline: 1.0083x; 1.0083x over previous
import jax
import jax.numpy as jnp
from jax import lax
from jax.experimental import pallas as pl
from jax.experimental.pallas import tpu as pltpu
from jax.experimental.pallas import tpu_sc as plsc

B = 16384
DIM = 64
PDIM = 128
NUM_CORES = 2
NUM_SUBCORES = 16
NW = NUM_CORES * NUM_SUBCORES
BPW = B // NW
CHUNK = 128
NCHUNK = BPW // CHUNK
L = 16
FGROUPS = DIM // L
BGROUPS = CHUNK // L


def _gather_body(rt_ref, ln_ref, tp_ref, w0_ref, w1_ref, w2_ref, out_ref,
                 i0a, i1a, i2a, i0b, i1b, i2b,
                 r0a, r1a, r2a, r0b, r1b, r2b,
                 outT, sem_a, sem_b):
    wid = lax.axis_index("s") * NUM_CORES + lax.axis_index("c")
    base = wid * BPW
    irefs = (rt_ref, ln_ref, tp_ref)
    tabs = (w0_ref, w1_ref, w2_ref)
    idxs = ((i0a, i1a, i2a), (i0b, i1b, i2b))
    rows = ((r0a, r1a, r2a), (r0b, r1b, r2b))
    sems = (sem_a, sem_b)

    iota = lax.iota(jnp.int32, L)
    perms = [lax.rem(iota + j, jnp.full((L,), L, jnp.int32)) for j in range(L)]

    def fire(k):
        s = k % 2
        b = base + k * CHUNK
        for c in range(3):
            pltpu.sync_copy(irefs[c].at[pl.ds(b, CHUNK)], idxs[s][c])
        return [pltpu.async_copy(tabs[c].at[idxs[s][c]], rows[s][c], sems[s])
                for c in range(3)]

    def transpose_block(src, outview, i):
        bg = lax.div(i, FGROUPS)
        fg = lax.rem(i, FGROUPS)
        ridx = bg * L + iota
        cidx0 = fg * L
        for j in range(L):
            cidx = cidx0 + perms[j]
            v = plsc.load_gather(src, [ridx, cidx])
            plsc.store_scatter(outview, [cidx, ridx], v)

    def process(k, descs):
        s = k % 2
        for d in descs:
            d.wait()
        for c in range(3):
            view = outT.at[pl.ds(DIM * c, DIM)]
            lax.fori_loop(
                0, BGROUPS * FGROUPS,
                lambda i, carry, c=c, view=view: (
                    transpose_block(rows[s][c], view, i), carry)[1],
                0)
        b = base + k * CHUNK
        pltpu.sync_copy(outT, out_ref.at[:, pl.ds(b, CHUNK)])

    descs = fire(0)
    for k in range(NCHUNK):
        nxt = fire(k + 1) if k + 1 < NCHUNK else None
        process(k, descs)
        descs = nxt


@jax.jit
def _lookup_concat(road_type, lane, time_period, W_road, W_lane, W_time):
    pad = [(0, 0), (0, PDIM - DIM)]
    w0 = jnp.pad(W_road, pad)
    w1 = jnp.pad(W_lane, pad)
    w2 = jnp.pad(W_time, pad)

    mesh = plsc.VectorSubcoreMesh(core_axis_name="c", subcore_axis_name="s")
    out_t = pl.kernel(
        _gather_body,
        out_type=jax.ShapeDtypeStruct((3 * DIM, B), jnp.float32),
        mesh=mesh,
        compiler_params=pltpu.CompilerParams(needs_layout_passes=False),
        scratch_types=[
            pltpu.VMEM((CHUNK,), jnp.int32),
            pltpu.VMEM((CHUNK,), jnp.int32),
            pltpu.VMEM((CHUNK,), jnp.int32),
            pltpu.VMEM((CHUNK,), jnp.int32),
            pltpu.VMEM((CHUNK,), jnp.int32),
            pltpu.VMEM((CHUNK,), jnp.int32),
            pltpu.VMEM((CHUNK, PDIM), jnp.float32),
            pltpu.VMEM((CHUNK, PDIM), jnp.float32),
            pltpu.VMEM((CHUNK, PDIM), jnp.float32),
            pltpu.VMEM((CHUNK, PDIM), jnp.float32),
            pltpu.VMEM((CHUNK, PDIM), jnp.float32),
            pltpu.VMEM((CHUNK, PDIM), jnp.float32),
            pltpu.VMEM((3 * DIM, CHUNK), jnp.float32),
            pltpu.SemaphoreType.DMA,
            pltpu.SemaphoreType.DMA,
        ],
    )(road_type, lane, time_period, w0, w1, w2)
    return out_t.T


def kernel(road_type, lane, time_period, W_road, W_lane, W_time):
    return _lookup_concat(
        road_type.astype(jnp.int32),
        lane.astype(jnp.int32),
        time_period.astype(jnp.int32),
        W_road, W_lane, W_time,
    )


# speedup vs baseline: 1.0223x; 1.0138x over previous
import jax
import jax.numpy as jnp
from jax import lax
from jax.experimental import pallas as pl
from jax.experimental.pallas import tpu as pltpu
from jax.experimental.pallas import tpu_sc as plsc

B = 16384
DIM = 64
PDIM = 128
NUM_CORES = 2
NUM_SUBCORES = 16
NW = NUM_CORES * NUM_SUBCORES
BPW = B // NW
CHUNK = 128
NCHUNK = BPW // CHUNK
L = 16
FGROUPS = DIM // L
BGROUPS = CHUNK // L


def _gather_body(rt_ref, ln_ref, tp_ref, w_ref, out_ref,
                 i0a, i1a, i2a, i0b, i1b, i2b,
                 r0a, r1a, r2a, r0b, r1b, r2b,
                 outT, sem_a, sem_b):
    wid = lax.axis_index("s") * NUM_CORES + lax.axis_index("c")
    base = wid * BPW
    irefs = (rt_ref, ln_ref, tp_ref)
    tabs = (w_ref, w_ref, w_ref)
    idxs = ((i0a, i1a, i2a), (i0b, i1b, i2b))
    rows = ((r0a, r1a, r2a), (r0b, r1b, r2b))
    sems = (sem_a, sem_b)

    iota = lax.iota(jnp.int32, L)
    perms = [lax.rem(iota + j, jnp.full((L,), L, jnp.int32)) for j in range(L)]

    def fire(k):
        s = k % 2
        b = base + k * CHUNK
        for c in range(3):
            pltpu.sync_copy(irefs[c].at[pl.ds(b, CHUNK)], idxs[s][c])
        return [pltpu.async_copy(tabs[c].at[idxs[s][c]], rows[s][c], sems[s])
                for c in range(3)]

    def transpose_block(src, outview, i):
        bg = lax.div(i, FGROUPS)
        fg = lax.rem(i, FGROUPS)
        ridx = bg * L + iota
        cidx0 = fg * L
        for j in range(L):
            cidx = cidx0 + perms[j]
            v = plsc.load_gather(src, [ridx, cidx])
            plsc.store_scatter(outview, [cidx, ridx], v)

    def process(k, descs):
        s = k % 2
        for d in descs:
            d.wait()
        for c in range(3):
            view = outT.at[pl.ds(DIM * c, DIM)]
            lax.fori_loop(
                0, BGROUPS * FGROUPS,
                lambda i, carry, c=c, view=view: (
                    transpose_block(rows[s][c], view, i), carry)[1],
                0)
        b = base + k * CHUNK
        pltpu.sync_copy(outT, out_ref.at[:, pl.ds(b, CHUNK)])

    descs = fire(0)
    for k in range(NCHUNK):
        nxt = fire(k + 1) if k + 1 < NCHUNK else None
        process(k, descs)
        descs = nxt


@jax.jit
def _lookup_concat(road_type, lane, time_period, W_road, W_lane, W_time):
    vocab = W_road.shape[0]
    w_all = jnp.pad(jnp.concatenate([W_road, W_lane, W_time], axis=0),
                    [(0, 0), (0, PDIM - DIM)])
    lane = lane + vocab
    time_period = time_period + 2 * vocab

    mesh = plsc.VectorSubcoreMesh(core_axis_name="c", subcore_axis_name="s")
    out_t = pl.kernel(
        _gather_body,
        out_type=jax.ShapeDtypeStruct((3 * DIM, B), jnp.float32),
        mesh=mesh,
        compiler_params=pltpu.CompilerParams(needs_layout_passes=False),
        scratch_types=[
            pltpu.VMEM((CHUNK,), jnp.int32),
            pltpu.VMEM((CHUNK,), jnp.int32),
            pltpu.VMEM((CHUNK,), jnp.int32),
            pltpu.VMEM((CHUNK,), jnp.int32),
            pltpu.VMEM((CHUNK,), jnp.int32),
            pltpu.VMEM((CHUNK,), jnp.int32),
            pltpu.VMEM((CHUNK, PDIM), jnp.float32),
            pltpu.VMEM((CHUNK, PDIM), jnp.float32),
            pltpu.VMEM((CHUNK, PDIM), jnp.float32),
            pltpu.VMEM((CHUNK, PDIM), jnp.float32),
            pltpu.VMEM((CHUNK, PDIM), jnp.float32),
            pltpu.VMEM((CHUNK, PDIM), jnp.float32),
            pltpu.VMEM((3 * DIM, CHUNK), jnp.float32),
            pltpu.SemaphoreType.DMA,
            pltpu.SemaphoreType.DMA,
        ],
    )(road_type, lane, time_period, w_all)
    return out_t.T


def kernel(road_type, lane, time_period, W_road, W_lane, W_time):
    return _lookup_concat(
        road_type.astype(jnp.int32),
        lane.astype(jnp.int32),
        time_period.astype(jnp.int32),
        W_road, W_lane, W_time,
    )


# speedup vs baseline: 1.4151x; 1.3843x over previous
import jax
import jax.numpy as jnp
from jax import lax
from jax.experimental import pallas as pl
from jax.experimental.pallas import tpu as pltpu
from jax.experimental.pallas import tpu_sc as plsc

B = 16384
DIM = 64
PDIM = 128
NUM_CORES = 2
NUM_SUBCORES = 16
NW = NUM_CORES * NUM_SUBCORES
BPW = B // NW
CHUNK = 128
NCHUNK = BPW // CHUNK
L = 16
FGROUPS = DIM // L
BGROUPS = CHUNK // L


def _gather_body(rt_ref, ln_ref, tp_ref, w_ref, out_ref,
                 i0a, i1a, i2a, i0b, i1b, i2b,
                 r0a, r1a, r2a, r0b, r1b, r2b,
                 outT, sem_a, sem_b):
    wid = lax.axis_index("s") * NUM_CORES + lax.axis_index("c")
    base = wid * BPW
    irefs = (rt_ref, ln_ref, tp_ref)
    tabs = (w_ref, w_ref, w_ref)
    idxs = ((i0a, i1a, i2a), (i0b, i1b, i2b))
    rows = ((r0a, r1a, r2a), (r0b, r1b, r2b))
    sems = (sem_a, sem_b)

    iota = lax.iota(jnp.int32, L)
    perms = [lax.rem(iota + j, jnp.full((L,), L, jnp.int32)) for j in range(L)]

    def fire(k):
        s = k % 2
        b = base + k * CHUNK
        for c in range(3):
            pltpu.sync_copy(irefs[c].at[pl.ds(b, CHUNK)], idxs[s][c])
        return [pltpu.async_copy(tabs[c].at[idxs[s][c]], rows[s][c], sems[s])
                for c in range(3)]

    def transpose_block(src, outview, i):
        bg = lax.div(i, FGROUPS)
        fg = lax.rem(i, FGROUPS)
        ridx = bg * L + iota
        cidx0 = fg * L
        for half in range(0, L, 8):
            vals = []
            for j in range(half, half + 8):
                vals.append(plsc.load_gather(src, [ridx, cidx0 + perms[j]]))
            for j in range(half, half + 8):
                plsc.store_scatter(outview, [cidx0 + perms[j], ridx],
                                   vals[j - half])

    def process(k, descs):
        s = k % 2
        for d in descs:
            d.wait()
        for c in range(3):
            view = outT.at[pl.ds(DIM * c, DIM)]
            lax.fori_loop(
                0, BGROUPS * FGROUPS,
                lambda i, carry, c=c, view=view: (
                    transpose_block(rows[s][c], view, i), carry)[1],
                0)
        b = base + k * CHUNK
        pltpu.sync_copy(outT, out_ref.at[:, pl.ds(b, CHUNK)])

    descs = fire(0)
    for k in range(NCHUNK):
        nxt = fire(k + 1) if k + 1 < NCHUNK else None
        process(k, descs)
        descs = nxt


@jax.jit
def _lookup_concat(road_type, lane, time_period, W_road, W_lane, W_time):
    vocab = W_road.shape[0]
    w_all = jnp.pad(jnp.concatenate([W_road, W_lane, W_time], axis=0),
                    [(0, 0), (0, PDIM - DIM)])
    lane = lane + vocab
    time_period = time_period + 2 * vocab

    mesh = plsc.VectorSubcoreMesh(core_axis_name="c", subcore_axis_name="s")
    out_t = pl.kernel(
        _gather_body,
        out_type=jax.ShapeDtypeStruct((3 * DIM, B), jnp.float32),
        mesh=mesh,
        compiler_params=pltpu.CompilerParams(needs_layout_passes=False),
        scratch_types=[
            pltpu.VMEM((CHUNK,), jnp.int32),
            pltpu.VMEM((CHUNK,), jnp.int32),
            pltpu.VMEM((CHUNK,), jnp.int32),
            pltpu.VMEM((CHUNK,), jnp.int32),
            pltpu.VMEM((CHUNK,), jnp.int32),
            pltpu.VMEM((CHUNK,), jnp.int32),
            pltpu.VMEM((CHUNK, PDIM), jnp.float32),
            pltpu.VMEM((CHUNK, PDIM), jnp.float32),
            pltpu.VMEM((CHUNK, PDIM), jnp.float32),
            pltpu.VMEM((CHUNK, PDIM), jnp.float32),
            pltpu.VMEM((CHUNK, PDIM), jnp.float32),
            pltpu.VMEM((CHUNK, PDIM), jnp.float32),
            pltpu.VMEM((3 * DIM, CHUNK), jnp.float32),
            pltpu.SemaphoreType.DMA,
            pltpu.SemaphoreType.DMA,
        ],
    )(road_type, lane, time_period, w_all)
    return out_t.T


def kernel(road_type, lane, time_period, W_road, W_lane, W_time):
    return _lookup_concat(
        road_type.astype(jnp.int32),
        lane.astype(jnp.int32),
        time_period.astype(jnp.int32),
        W_road, W_lane, W_time,
    )


# speedup vs baseline: 1.4154x; 1.0002x over previous
import jax
import jax.numpy as jnp
from jax import lax
from jax.experimental import pallas as pl
from jax.experimental.pallas import tpu as pltpu
from jax.experimental.pallas import tpu_sc as plsc

B = 16384
DIM = 64
PDIM = 128
NUM_CORES = 2
NUM_SUBCORES = 16
NW = NUM_CORES * NUM_SUBCORES
BPW = B // NW
CHUNK = 128
NCHUNK = BPW // CHUNK
L = 16
FGROUPS = DIM // L
BGROUPS = CHUNK // L


def _gather_body(rt_ref, ln_ref, tp_ref, w_ref, out_ref,
                 i0a, i1a, i2a, i0b, i1b, i2b,
                 r0a, r1a, r2a, r0b, r1b, r2b,
                 outT, sem_a, sem_b):
    wid = lax.axis_index("s") * NUM_CORES + lax.axis_index("c")
    base = wid * BPW
    irefs = (rt_ref, ln_ref, tp_ref)
    tabs = (w_ref, w_ref, w_ref)
    idxs = ((i0a, i1a, i2a), (i0b, i1b, i2b))
    rows = ((r0a, r1a, r2a), (r0b, r1b, r2b))
    sems = (sem_a, sem_b)

    iota = lax.iota(jnp.int32, L)
    perms = [lax.rem(iota + j, jnp.full((L,), L, jnp.int32)) for j in range(L)]

    def fire(k):
        s = k % 2
        b = base + k * CHUNK
        for c in range(3):
            pltpu.sync_copy(irefs[c].at[pl.ds(b, CHUNK)], idxs[s][c])
        return [pltpu.async_copy(tabs[c].at[idxs[s][c]], rows[s][c], sems[s])
                for c in range(3)]

    def transpose_block(src, outview, i):
        bg = lax.div(i, FGROUPS)
        fg = lax.rem(i, FGROUPS)
        ridx = bg * L + iota
        cidx0 = fg * L
        vals = []
        for j in range(L):
            vals.append(plsc.load_gather(src, [ridx, cidx0 + perms[j]]))
        for j in range(L):
            plsc.store_scatter(outview, [cidx0 + perms[j], ridx], vals[j])

    def process(k, descs):
        s = k % 2
        for d in descs:
            d.wait()
        for c in range(3):
            view = outT.at[pl.ds(DIM * c, DIM)]
            lax.fori_loop(
                0, BGROUPS * FGROUPS,
                lambda i, carry, c=c, view=view: (
                    transpose_block(rows[s][c], view, i), carry)[1],
                0)
        b = base + k * CHUNK
        pltpu.sync_copy(outT, out_ref.at[:, pl.ds(b, CHUNK)])

    descs = fire(0)
    for k in range(NCHUNK):
        nxt = fire(k + 1) if k + 1 < NCHUNK else None
        process(k, descs)
        descs = nxt


@jax.jit
def _lookup_concat(road_type, lane, time_period, W_road, W_lane, W_time):
    vocab = W_road.shape[0]
    w_all = jnp.pad(jnp.concatenate([W_road, W_lane, W_time], axis=0),
                    [(0, 0), (0, PDIM - DIM)])
    lane = lane + vocab
    time_period = time_period + 2 * vocab

    mesh = plsc.VectorSubcoreMesh(core_axis_name="c", subcore_axis_name="s")
    out_t = pl.kernel(
        _gather_body,
        out_type=jax.ShapeDtypeStruct((3 * DIM, B), jnp.float32),
        mesh=mesh,
        compiler_params=pltpu.CompilerParams(needs_layout_passes=False),
        scratch_types=[
            pltpu.VMEM((CHUNK,), jnp.int32),
            pltpu.VMEM((CHUNK,), jnp.int32),
            pltpu.VMEM((CHUNK,), jnp.int32),
            pltpu.VMEM((CHUNK,), jnp.int32),
            pltpu.VMEM((CHUNK,), jnp.int32),
            pltpu.VMEM((CHUNK,), jnp.int32),
            pltpu.VMEM((CHUNK, PDIM), jnp.float32),
            pltpu.VMEM((CHUNK, PDIM), jnp.float32),
            pltpu.VMEM((CHUNK, PDIM), jnp.float32),
            pltpu.VMEM((CHUNK, PDIM), jnp.float32),
            pltpu.VMEM((CHUNK, PDIM), jnp.float32),
            pltpu.VMEM((CHUNK, PDIM), jnp.float32),
            pltpu.VMEM((3 * DIM, CHUNK), jnp.float32),
            pltpu.SemaphoreType.DMA,
            pltpu.SemaphoreType.DMA,
        ],
    )(road_type, lane, time_period, w_all)
    return out_t.T


def kernel(road_type, lane, time_period, W_road, W_lane, W_time):
    return _lookup_concat(
        road_type.astype(jnp.int32),
        lane.astype(jnp.int32),
        time_period.astype(jnp.int32),
        W_road, W_lane, W_time,
    )
